# baseline (speedup 1.0000x reference)
import jax
import jax.numpy as jnp
from jax.experimental import pallas as pl
from jax.experimental.pallas import tpu as pltpu


def kernel(x, idxs):
    B, C, T, H, W = x.shape
    BC = B * C
    HW = H * W
    xr = x.reshape(BC, T, HW)
    idxs32 = idxs.astype(jnp.int32)

    BC_BLK = 256
    NI = BC // BC_BLK
    NCH = NI * T
    NBUF = 8
    LOOK = 4

    def body(idx_ref, x_hbm, o_hbm, buf, insems, outsems):
        def in_start(k, slot):
            i = k // T
            t = jax.lax.rem(k, T)
            src = idx_ref[t]
            pltpu.make_async_copy(
                x_hbm.at[pl.ds(i * BC_BLK, BC_BLK), src],
                buf.at[slot],
                insems.at[slot],
            ).start()

        def in_wait(k, slot):
            i = k // T
            pltpu.make_async_copy(
                x_hbm.at[pl.ds(i * BC_BLK, BC_BLK), 0],
                buf.at[slot],
                insems.at[slot],
            ).wait()

        def out_start(k, slot):
            i = k // T
            t = jax.lax.rem(k, T)
            pltpu.make_async_copy(
                buf.at[slot],
                o_hbm.at[pl.ds(i * BC_BLK, BC_BLK), t],
                outsems.at[slot],
            ).start()

        def out_wait(k, slot):
            i = k // T
            pltpu.make_async_copy(
                buf.at[slot],
                o_hbm.at[pl.ds(i * BC_BLK, BC_BLK), 0],
                outsems.at[slot],
            ).wait()

        for j in range(LOOK):
            in_start(j, j)

        @pl.loop(0, NCH, step=NBUF)
        def _gen(ko):
            for b in range(NBUF):
                k = ko + b
                slot_next = (b + LOOK) % NBUF

                @pl.when(k + LOOK < NCH)
                def _():
                    @pl.when(k + LOOK >= NBUF)
                    def _():
                        out_wait(k + LOOK - NBUF, slot_next)

                    in_start(k + LOOK, slot_next)

                in_wait(k, b)
                out_start(k, b)

        for b in range(NBUF):
            out_wait(NCH - NBUF + b, b)

    out = pl.pallas_call(
        body,
        grid_spec=pltpu.PrefetchScalarGridSpec(
            num_scalar_prefetch=1,
            grid=(1,),
            in_specs=[pl.BlockSpec(memory_space=pl.ANY)],
            out_specs=pl.BlockSpec(memory_space=pl.ANY),
            scratch_shapes=[
                pltpu.VMEM((NBUF, BC_BLK, HW), jnp.float32),
                pltpu.SemaphoreType.DMA((NBUF,)),
                pltpu.SemaphoreType.DMA((NBUF,)),
            ],
        ),
        out_shape=jax.ShapeDtypeStruct((BC, T, HW), x.dtype),
    )(idxs32, xr)
    return out.reshape(B, C, T, H, W)


# speedup vs baseline: 1.6476x; 1.6476x over previous
import jax
import jax.numpy as jnp
from jax.experimental import pallas as pl
from jax.experimental.pallas import tpu as pltpu


def kernel(x, idxs):
    B, C, T, H, W = x.shape
    BC = B * C
    xr = x.reshape(BC, T, H, W)
    idxs32 = idxs.astype(jnp.int32)

    BC_BLK = 128
    NI = BC // BC_BLK
    NCH = NI * T
    NBUF = 8
    LOOK = 4

    def body(idx_ref, x_hbm, o_hbm, buf, insems, outsems):
        def in_start(k, slot):
            i = k // T
            t = jax.lax.rem(k, T)
            src = idx_ref[t]
            pltpu.make_async_copy(
                x_hbm.at[pl.ds(i * BC_BLK, BC_BLK), src],
                buf.at[slot],
                insems.at[slot],
            ).start()

        def in_wait(k, slot):
            i = k // T
            pltpu.make_async_copy(
                x_hbm.at[pl.ds(i * BC_BLK, BC_BLK), 0],
                buf.at[slot],
                insems.at[slot],
            ).wait()

        def out_start(k, slot):
            i = k // T
            t = jax.lax.rem(k, T)
            pltpu.make_async_copy(
                buf.at[slot],
                o_hbm.at[pl.ds(i * BC_BLK, BC_BLK), t],
                outsems.at[slot],
            ).start()

        def out_wait(k, slot):
            i = k // T
            pltpu.make_async_copy(
                buf.at[slot],
                o_hbm.at[pl.ds(i * BC_BLK, BC_BLK), 0],
                outsems.at[slot],
            ).wait()

        for j in range(LOOK):
            in_start(j, j)

        @pl.loop(0, NCH, step=NBUF)
        def _gen(ko):
            for b in range(NBUF):
                k = ko + b
                slot_next = (b + LOOK) % NBUF

                @pl.when(k + LOOK < NCH)
                def _():
                    @pl.when(k + LOOK >= NBUF)
                    def _():
                        out_wait(k + LOOK - NBUF, slot_next)

                    in_start(k + LOOK, slot_next)

                in_wait(k, b)
                out_start(k, b)

        for b in range(NBUF):
            out_wait(NCH - NBUF + b, b)

    out = pl.pallas_call(
        body,
        grid_spec=pltpu.PrefetchScalarGridSpec(
            num_scalar_prefetch=1,
            grid=(1,),
            in_specs=[pl.BlockSpec(memory_space=pl.ANY)],
            out_specs=pl.BlockSpec(memory_space=pl.ANY),
            scratch_shapes=[
                pltpu.VMEM((NBUF, BC_BLK, H, W), jnp.float32),
                pltpu.SemaphoreType.DMA((NBUF,)),
                pltpu.SemaphoreType.DMA((NBUF,)),
            ],
        ),
        out_shape=jax.ShapeDtypeStruct((BC, T, H, W), x.dtype),
    )(idxs32, xr)
    return out.reshape(B, C, T, H, W)
